# Initial kernel scaffold; baseline (speedup 1.0000x reference)
import jax
import jax.numpy as jnp
from jax.experimental import pallas as pl


def kernel(x_cat, x_cont, hist_seq, cat_tables, seq_table, W1, b1, W2, b2):
    raise NotImplementedError("write your pallas kernel here")


# speedup vs baseline: 1.9420x; 1.9420x over previous
import functools

import jax
import jax.numpy as jnp
from jax import lax
from jax.experimental import pallas as pl
from jax.experimental.pallas import tpu as pltpu
from jax.experimental.pallas import tpu_sc as plsc

B = 4096
NCAT = 26
VCAT = 100000
VSEQ = 1000000
L = 50
D = 32
NCONT = 13
HID = 128

NC = 2
NS = 16
NW = NC * NS
BPW = B // NW
CHUNK = 16
NCHUNK = BPW // CHUNK
CATN = CHUNK * NCAT
SEQN = CHUNK * L


def _sc_body(xcat_hbm, hist_hbm, cat_tab_hbm, seq_tab_hbm,
             catrows_out, pooled_out,
             offs_v, xcat_v, idx_v, hist_v, catrows_v, seqrows_v, pooled_v,
             sem):
  wid = lax.axis_index("s") * NC + lax.axis_index("c")
  base = wid * BPW

  for j in range(NCAT):
    pos = lax.iota(jnp.int32, 16) + 16 * j
    offs_v[pl.ds(16 * j, 16)] = lax.rem(pos, NCAT) * VCAT

  for c in range(NCHUNK):
    b0 = base + c * CHUNK
    d1 = pltpu.make_async_copy(
        xcat_hbm.at[pl.ds(b0 * NCAT, CATN)], xcat_v, sem)
    d1.start()
    d2 = pltpu.make_async_copy(
        hist_hbm.at[pl.ds(b0 * L, SEQN)], hist_v, sem)
    d2.start()
    d1.wait()
    d2.wait()

    for j in range(NCAT):
      s = pl.ds(16 * j, 16)
      idx_v[s] = xcat_v[s] + offs_v[s]

    descs = []
    for g in range(3):
      descs.append(pltpu.make_async_copy(
          cat_tab_hbm.at[idx_v.at[pl.ds(128 * g, 128)]],
          catrows_v.at[pl.ds(128 * g, 128)], sem))
    descs.append(pltpu.make_async_copy(
        cat_tab_hbm.at[idx_v.at[pl.ds(384, 32)]],
        catrows_v.at[pl.ds(384, 32)], sem))
    for g in range(6):
      descs.append(pltpu.make_async_copy(
          seq_tab_hbm.at[hist_v.at[pl.ds(128 * g, 128)]],
          seqrows_v.at[pl.ds(128 * g, 128)], sem))
    descs.append(pltpu.make_async_copy(
        seq_tab_hbm.at[hist_v.at[pl.ds(768, 32)]],
        seqrows_v.at[pl.ds(768, 32)], sem))
    for d in descs:
      d.start()
    for d in descs:
      d.wait()

    def pool_b(b, carry):
      def pool_l(t, accs):
        a0, a1 = accs
        r = b * L + t * 5
        for u in range(5):
          a0 = a0 + seqrows_v[r + u, pl.ds(0, 16)]
          a1 = a1 + seqrows_v[r + u, pl.ds(16, 16)]
        return (a0, a1)
      z = jnp.zeros((16,), jnp.float32)
      a0, a1 = lax.fori_loop(0, L // 5, pool_l, (z, z))
      pooled_v[b, pl.ds(0, 16)] = a0 * (1.0 / L)
      pooled_v[b, pl.ds(16, 16)] = a1 * (1.0 / L)
      return carry

    lax.fori_loop(0, CHUNK, pool_b, 0)

    pltpu.sync_copy(catrows_v, catrows_out.at[pl.ds(b0 * NCAT, CATN)])
    pltpu.sync_copy(pooled_v, pooled_out.at[pl.ds(b0, CHUNK)])


def _mlp_body(x1_ref, xc_ref, xp_ref, w1a_ref, w1b_ref, w1c_ref,
              b1_ref, w2_ref, b2_ref, out_ref):
  h = jnp.dot(x1_ref[...], w1a_ref[...], preferred_element_type=jnp.float32)
  h = h + jnp.dot(xc_ref[...], w1b_ref[...],
                  preferred_element_type=jnp.float32)
  h = h + jnp.dot(xp_ref[...], w1c_ref[...],
                  preferred_element_type=jnp.float32)
  h = jax.nn.relu(h + b1_ref[...])
  out = jnp.dot(h, w2_ref[...], preferred_element_type=jnp.float32)
  out_ref[...] = out + b2_ref[0, 0]


def kernel(x_cat, x_cont, hist_seq, cat_tables, seq_table, W1, b1, W2, b2):
  xcat_flat = x_cat.reshape(-1)
  hist_flat = hist_seq.reshape(-1)
  cat_tab = cat_tables.reshape(NCAT * VCAT, D)

  mesh = plsc.VectorSubcoreMesh(core_axis_name="c", subcore_axis_name="s")
  sc = pl.kernel(
      _sc_body,
      out_type=(
          jax.ShapeDtypeStruct((B * NCAT, D), jnp.float32),
          jax.ShapeDtypeStruct((B, D), jnp.float32),
      ),
      mesh=mesh,
      compiler_params=pltpu.CompilerParams(use_tc_tiling_on_sc=False),
      scratch_types=[
          pltpu.VMEM((CATN,), jnp.int32),
          pltpu.VMEM((CATN,), jnp.int32),
          pltpu.VMEM((CATN,), jnp.int32),
          pltpu.VMEM((SEQN,), jnp.int32),
          pltpu.VMEM((CATN, D), jnp.float32),
          pltpu.VMEM((SEQN, D), jnp.float32),
          pltpu.VMEM((CHUNK, D), jnp.float32),
          pltpu.SemaphoreType.DMA,
      ],
  )
  catrows, pooled = sc(xcat_flat, hist_flat, cat_tab, seq_table)
  cat_flat = catrows.reshape(B, NCAT * D)

  w1a = W1[: NCAT * D]
  w1b = W1[NCAT * D: NCAT * D + NCONT]
  w1c = W1[NCAT * D + NCONT:]
  b1r = b1.reshape(1, HID)
  b2r = b2.reshape(1, 1)

  bm = 512
  grid = (B // bm,)
  logits = pl.pallas_call(
      _mlp_body,
      grid=grid,
      in_specs=[
          pl.BlockSpec((bm, NCAT * D), lambda i: (i, 0)),
          pl.BlockSpec((bm, NCONT), lambda i: (i, 0)),
          pl.BlockSpec((bm, D), lambda i: (i, 0)),
          pl.BlockSpec((NCAT * D, HID), lambda i: (0, 0)),
          pl.BlockSpec((NCONT, HID), lambda i: (0, 0)),
          pl.BlockSpec((D, HID), lambda i: (0, 0)),
          pl.BlockSpec((1, HID), lambda i: (0, 0)),
          pl.BlockSpec((HID, 1), lambda i: (0, 0)),
          pl.BlockSpec((1, 1), lambda i: (0, 0)),
      ],
      out_specs=pl.BlockSpec((bm, 1), lambda i: (i, 0)),
      out_shape=jax.ShapeDtypeStruct((B, 1), jnp.float32),
  )(cat_flat, x_cont, pooled, w1a, w1b, w1c, b1r, W2, b2r)
  return logits.reshape(B)
